# baseline (speedup 1.0000x reference)
import functools

import jax
import jax.numpy as jnp
from jax import lax
from jax.experimental import pallas as pl
from jax.experimental.pallas import tpu as pltpu
from jax.experimental.pallas import tpu_sc as plsc

EMB_D = 32
NB = 8
LOOKAHEAD = 4


def _make_gather(b0: int, b1: int):
    info = plsc.get_sparse_core_info()
    nw = info.num_cores * info.num_subcores
    rows_per_w = b0 // nw
    n_groups = rows_per_w // NB
    assert b0 % nw == 0 and rows_per_w % NB == 0 and n_groups >= 3
    mesh = plsc.VectorSubcoreMesh(core_axis_name="c", subcore_axis_name="s")

    @functools.partial(
        pl.kernel,
        mesh=mesh,
        out_type=jax.ShapeDtypeStruct((b0, b1, EMB_D), jnp.float32),
        scratch_types=[
            pltpu.VMEM((rows_per_w, 128), jnp.int32),
            pltpu.VMEM((NB, 128, EMB_D), jnp.float32),
            pltpu.SemaphoreType.DMA((NB,)),
            pltpu.SemaphoreType.DMA((NB,)),
        ],
        compiler_params=pltpu.CompilerParams(use_tc_tiling_on_sc=False),
    )
    def gather(idx_hbm, table_hbm, out_hbm, idx_v, rows_v, gsem, ssem):
        wid = lax.axis_index("s") * info.num_cores + lax.axis_index("c")
        base = wid * rows_per_w
        pltpu.sync_copy(idx_hbm.at[pl.ds(base, rows_per_w)], idx_v)

        def g_desc(i, b):
            return pltpu.make_async_copy(
                table_hbm.at[idx_v.at[i]], rows_v.at[b], gsem.at[b])

        def s_desc(i, b):
            return pltpu.make_async_copy(
                rows_v.at[b, pl.ds(0, b1)], out_hbm.at[base + i],
                ssem.at[b])

        for b in range(LOOKAHEAD):
            g_desc(b, b).start()

        def step(i, b, first_group: bool, launch: bool):
            if launch:
                j = i + LOOKAHEAD
                q = (b + LOOKAHEAD) % NB
                if not (first_group and b + LOOKAHEAD < NB):
                    s_desc(j - NB, q).wait()
                g_desc(j, q).start()
            g_desc(i, b).wait()
            s_desc(i, b).start()

        for b in range(NB):
            step(b, b, first_group=True, launch=True)

        def group(g, carry):
            for b in range(NB):
                step(g * NB + b, b, first_group=False, launch=True)
            return carry

        lax.fori_loop(1, n_groups - 1, group, 0)

        last = (n_groups - 1) * NB
        for b in range(NB):
            step(last + b, b, first_group=False,
                 launch=(last + b + LOOKAHEAD < rows_per_w))

        for b in range(NB):
            s_desc(rows_per_w - NB + b, b).wait()

    return gather


def kernel(IX, weight):
    b0, b1 = IX.shape
    ixp = jnp.pad(IX.astype(jnp.int32), ((0, 0), (0, 128 - b1)))
    return _make_gather(b0, b1)(ixp, weight)


# speedup vs baseline: 12.7212x; 12.7212x over previous
import functools

import jax
import jax.numpy as jnp
from jax import lax
from jax.experimental import pallas as pl
from jax.experimental.pallas import tpu as pltpu
from jax.experimental.pallas import tpu_sc as plsc

EMB_D = 32
SB = 4
NB = 2 * SB
LOOKAHEAD = SB


def _make_gather(b0: int, b1: int):
    info = plsc.get_sparse_core_info()
    nw = info.num_cores * info.num_subcores
    rows_per_w = b0 // nw
    n_chunks = rows_per_w // SB
    assert b0 % nw == 0 and rows_per_w % SB == 0 and n_chunks >= 3
    mesh = plsc.VectorSubcoreMesh(core_axis_name="c", subcore_axis_name="s")

    @functools.partial(
        pl.kernel,
        mesh=mesh,
        out_type=jax.ShapeDtypeStruct((b0, b1, EMB_D), jnp.float32),
        scratch_types=[
            pltpu.VMEM((rows_per_w, b1), jnp.int32),
            pltpu.VMEM((NB, b1, EMB_D), jnp.float32),
            pltpu.SemaphoreType.DMA((NB,)),
            pltpu.SemaphoreType.DMA((2,)),
        ],
        compiler_params=pltpu.CompilerParams(use_tc_tiling_on_sc=False),
    )
    def gather(idx_hbm, table_hbm, out_hbm, idx_v, rows_v, gsem, ssem):
        wid = lax.axis_index("s") * info.num_cores + lax.axis_index("c")
        base = wid * rows_per_w
        pltpu.sync_copy(idx_hbm.at[pl.ds(base, rows_per_w)], idx_v)

        def g_desc(i, b):
            return pltpu.make_async_copy(
                table_hbm.at[idx_v.at[i]], rows_v.at[b], gsem.at[b])

        def s_desc(c, h):
            return pltpu.make_async_copy(
                rows_v.at[pl.ds(h * SB, SB)],
                out_hbm.at[pl.ds(base + c * SB, SB)], ssem.at[h])

        for u in range(SB):
            g_desc(u, u).start()

        for u in range(SB):
            g_desc(SB + u, SB + u).start()
            g_desc(u, u).wait()
        s_desc(0, 0).start()

        def chunk(c, carry):
            h = lax.rem(c, 2)
            hn = lax.rem(c + 1, 2)
            for u in range(SB):
                if u == 0:
                    s_desc(c - 1, hn).wait()
                g_desc((c + 1) * SB + u, hn * SB + u).start()
                g_desc(c * SB + u, h * SB + u).wait()
            s_desc(c, h).start()
            return carry

        lax.fori_loop(1, n_chunks - 1, chunk, 0)

        hl = (n_chunks - 1) % 2
        for u in range(SB):
            g_desc((n_chunks - 1) * SB + u, hl * SB + u).wait()
        s_desc(n_chunks - 1, hl).start()

        s_desc(n_chunks - 2, (n_chunks - 2) % 2).wait()
        s_desc(n_chunks - 1, hl).wait()

    return gather


def kernel(IX, weight):
    b0, b1 = IX.shape
    return _make_gather(b0, b1)(IX.astype(jnp.int32), weight)


# speedup vs baseline: 12.7469x; 1.0020x over previous
import functools

import jax
import jax.numpy as jnp
from jax import lax
from jax.experimental import pallas as pl
from jax.experimental.pallas import tpu as pltpu
from jax.experimental.pallas import tpu_sc as plsc

EMB_D = 32
NB = 8
LOOKAHEAD = 4


def _make_gather(b0: int, b1: int):
    info = plsc.get_sparse_core_info()
    nw = info.num_cores * info.num_subcores
    rows_per_w = b0 // nw
    n_groups = rows_per_w // NB
    assert b0 % nw == 0 and rows_per_w % NB == 0 and n_groups >= 3
    mesh = plsc.VectorSubcoreMesh(core_axis_name="c", subcore_axis_name="s")

    @functools.partial(
        pl.kernel,
        mesh=mesh,
        out_type=jax.ShapeDtypeStruct((b0, b1, EMB_D), jnp.float32),
        scratch_types=[
            pltpu.VMEM((rows_per_w, b1), jnp.int32),
            pltpu.VMEM((NB, b1, EMB_D), jnp.float32),
            pltpu.SemaphoreType.DMA((NB,)),
            pltpu.SemaphoreType.DMA((NB,)),
        ],
        compiler_params=pltpu.CompilerParams(use_tc_tiling_on_sc=False),
    )
    def gather(idx_hbm, table_hbm, out_hbm, idx_v, rows_v, gsem, ssem):
        wid = lax.axis_index("s") * info.num_cores + lax.axis_index("c")
        base = wid * rows_per_w
        pltpu.sync_copy(idx_hbm.at[pl.ds(base, rows_per_w)], idx_v)

        def g_desc(i, b):
            return pltpu.make_async_copy(
                table_hbm.at[idx_v.at[i]], rows_v.at[b], gsem.at[b])

        def s_desc(i, b):
            return pltpu.make_async_copy(
                rows_v.at[b], out_hbm.at[base + i], ssem.at[b])

        for b in range(LOOKAHEAD):
            g_desc(b, b).start()

        def step(i, b, first_group: bool, launch: bool):
            if launch:
                j = i + LOOKAHEAD
                q = (b + LOOKAHEAD) % NB
                if not (first_group and b + LOOKAHEAD < NB):
                    s_desc(j - NB, q).wait()
                g_desc(j, q).start()
            g_desc(i, b).wait()
            s_desc(i, b).start()

        for b in range(NB):
            step(b, b, first_group=True, launch=True)

        def group(g, carry):
            for b in range(NB):
                step(g * NB + b, b, first_group=False, launch=True)
            return carry

        lax.fori_loop(1, n_groups - 1, group, 0)

        last = (n_groups - 1) * NB
        for b in range(NB):
            step(last + b, b, first_group=False,
                 launch=(last + b + LOOKAHEAD < rows_per_w))

        for b in range(NB):
            s_desc(rows_per_w - NB + b, b).wait()

    return gather


def kernel(IX, weight):
    b0, b1 = IX.shape
    return _make_gather(b0, b1)(IX.astype(jnp.int32), weight)
